# baseline (speedup 1.0000x reference)
import math

import jax
import jax.numpy as jnp
from jax.experimental import pallas as pl
from jax.experimental.pallas import tpu as pltpu

_TARGET_SNR = 0.3
_EPS = 1e-12


def _stage1_body(x_ref, mask_ref, qn_ref, rms_ref, li_ref):
    B, S, D = x_ref.shape
    q_rows = []
    rms_rows = []
    li_rows = []
    iota_s = jax.lax.broadcasted_iota(jnp.int32, (1, S), 1)
    for b in range(B):
        xb = x_ref[b]
        mb = mask_ref[pl.ds(b, 1), :]
        cnt = jnp.sum(mb, axis=1, keepdims=True)
        denom = jnp.maximum(cnt, 1.0)
        li_b = (jnp.maximum(cnt, 1.0) - 1.0).astype(jnp.int32)
        qsum = jax.lax.dot_general(
            mb, xb, (((1,), (0,)), ((), ())),
            preferred_element_type=jnp.float32)
        qmean = qsum / denom
        qn = qmean / jnp.maximum(
            jnp.sqrt(jnp.sum(qmean * qmean, axis=1, keepdims=True)), _EPS)
        q_rows.append(qn.astype(jnp.bfloat16))
        onehot = (iota_s == li_b).astype(jnp.float32)
        host = jax.lax.dot_general(
            onehot, xb, (((1,), (0,)), ((), ())),
            preferred_element_type=jnp.float32)
        rms = jnp.sqrt(jnp.mean(host * host, axis=1, keepdims=True))
        rms_rows.append(jnp.broadcast_to(rms, (1, 128)))
        li_rows.append(jnp.broadcast_to(li_b, (1, 128)))
    qn_ref[...] = jnp.concatenate(q_rows, axis=0)
    rms_ref[...] = jnp.concatenate(rms_rows, axis=0)
    li_ref[...] = jnp.concatenate(li_rows, axis=0)


def _stage2_body(k_ref, kn_ref):
    k = k_ref[...]
    norm = jnp.sqrt(jnp.sum(k * k, axis=1, keepdims=True))
    kn_ref[...] = (k / jnp.maximum(norm, _EPS)).astype(jnp.bfloat16)


def _stage3_body(ki_ref, kj_ref, q_ref, nmax_ref, sims_ref, *, nkb):
    i = pl.program_id(0)
    j = pl.program_id(1)
    g = jax.lax.dot_general(
        ki_ref[...], kj_ref[...], (((1,), (1,)), ((), ())),
        preferred_element_type=jnp.float32)
    KB = g.shape[0]
    rows = jax.lax.broadcasted_iota(jnp.int32, (KB, KB), 0)
    cols = jax.lax.broadcasted_iota(jnp.int32, (KB, KB), 1)
    on_diag = jnp.logical_and(rows == cols, i == j)
    g = jnp.where(on_diag, -jnp.inf, g)
    pmax = jnp.max(g, axis=1, keepdims=True)

    @pl.when(j == 0)
    def _init():
        nmax_ref[...] = pmax

    @pl.when(j != 0)
    def _acc():
        nmax_ref[...] = jnp.maximum(nmax_ref[...], pmax)

    @pl.when(i == 0)
    def _sims():
        sims_ref[...] = jax.lax.dot_general(
            q_ref[...], kj_ref[...], (((1,), (1,)), ((), ())),
            preferred_element_type=jnp.float32)


def _kth_largest_rows(a, k, iters=40):
    rmin = jnp.min(a, axis=1, keepdims=True)
    rmax = jnp.max(a, axis=1, keepdims=True)
    lo0 = rmin - 1.0
    hi0 = rmax + 1.0

    def body(_, carry):
        lo, hi = carry
        mid = 0.5 * (lo + hi)
        cnt = jnp.sum((a >= mid).astype(jnp.float32), axis=1, keepdims=True)
        ge = cnt >= float(k)
        return (jnp.where(ge, mid, lo), jnp.where(ge, hi, mid))

    lo, hi = jax.lax.fori_loop(0, iters, body, (lo0, hi0))
    return jnp.max(jnp.where(a < hi, a, rmin - 2.0), axis=1, keepdims=True)


def _stage4_body(sims_ref, nm_ref, rms_ref, w_ref, gs_ref, *,
                 nk, d, eff_k, temperature):
    sims = sims_ref[...]
    nm = nm_ref[...]
    sqrt_d = math.sqrt(float(d))

    mean_raw = jnp.mean(sims, axis=1, keepdims=True)
    var_raw = jnp.mean((sims - mean_raw) ** 2, axis=1, keepdims=True)
    std_raw = jnp.maximum(jnp.sqrt(var_raw), 1e-6)
    max_raw = jnp.max(sims, axis=1, keepdims=True)
    z_peak = (max_raw - mean_raw) / std_raw
    gate_peak = jax.nn.sigmoid(z_peak * sqrt_d)

    nm_flat = nm.reshape(1, -1)
    k1 = (nk + 1) // 2
    k2 = nk // 2 + 1
    v1 = _kth_largest_rows(nm_flat, k1)
    v2 = _kth_largest_rows(nm_flat, k2)
    tau = 0.5 * (v1[0, 0] + v2[0, 0])
    mu_nm = jnp.mean(nm_flat)
    sigma = jnp.maximum(jnp.sqrt(jnp.mean((nm_flat - mu_nm) ** 2)), 1e-6)
    gate_manifold = jax.nn.sigmoid((max_raw - tau) / sigma * sqrt_d)

    if eff_k < nk:
        thr = _kth_largest_rows(sims, eff_k)
        logits = jnp.where(sims >= thr, sims, -1e9)
    else:
        logits = sims
    ex = jnp.exp((logits - max_raw) / temperature)
    w_ref[...] = ex / jnp.sum(ex, axis=1, keepdims=True)

    gate = gate_peak * gate_manifold
    magnitude = rms_ref[...] * _TARGET_SNR
    gs_ref[...] = gate * magnitude


def _stage5_body(w_ref, v_ref, acc_ref):
    j = pl.program_id(0)

    @pl.when(j == 0)
    def _init():
        acc_ref[...] = jnp.zeros_like(acc_ref)

    part = jax.lax.dot_general(
        w_ref[...].astype(jnp.bfloat16), v_ref[...].astype(jnp.bfloat16),
        (((1,), (0,)), ((), ())), preferred_element_type=jnp.float32)
    acc_ref[...] = acc_ref[...] + part


def _stage6_body(li_ref, x_ref, ret_ref, gs_ref, out_ref):
    r = ret_ref[...]
    norm = jnp.sqrt(jnp.sum(r * r, axis=2, keepdims=True))
    dirn = r / jnp.maximum(norm, _EPS)
    out_ref[...] = x_ref[...] + gs_ref[0, 0, 0] * dirn


def kernel(x, attention_mask, keys, values):
    B, S, D = x.shape
    NK = keys.shape[0]
    temperature = math.sqrt(math.log1p(float(NK))) / max(float(D), 1.0)
    eff_k = min(NK, max(2, int(math.ceil(
        math.sqrt(float(NK)) * math.log1p(float(NK))))))

    mask_f = attention_mask.astype(jnp.float32)

    qn, rmsarr, liarr = pl.pallas_call(
        _stage1_body,
        out_shape=[
            jax.ShapeDtypeStruct((B, D), jnp.bfloat16),
            jax.ShapeDtypeStruct((B, 128), jnp.float32),
            jax.ShapeDtypeStruct((B, 128), jnp.int32),
        ],
    )(x, mask_f)

    KB = 1024
    nkb = NK // KB
    kn = pl.pallas_call(
        _stage2_body,
        grid=(nkb,),
        in_specs=[pl.BlockSpec((KB, D), lambda i: (i, 0))],
        out_specs=pl.BlockSpec((KB, D), lambda i: (i, 0)),
        out_shape=jax.ShapeDtypeStruct((NK, D), jnp.bfloat16),
    )(keys)

    import functools
    nmax, sims = pl.pallas_call(
        functools.partial(_stage3_body, nkb=nkb),
        grid=(nkb, nkb),
        in_specs=[
            pl.BlockSpec((KB, D), lambda i, j: (i, 0)),
            pl.BlockSpec((KB, D), lambda i, j: (j, 0)),
            pl.BlockSpec((B, D), lambda i, j: (0, 0)),
        ],
        out_specs=[
            pl.BlockSpec((KB, 1), lambda i, j: (i, 0)),
            pl.BlockSpec((B, KB), lambda i, j: (0, j)),
        ],
        out_shape=[
            jax.ShapeDtypeStruct((NK, 1), jnp.float32),
            jax.ShapeDtypeStruct((B, NK), jnp.float32),
        ],
    )(kn, kn, qn)

    nm2d = nmax.reshape(NK // 128, 128)
    weights, gscale = pl.pallas_call(
        functools.partial(_stage4_body, nk=NK, d=D, eff_k=eff_k,
                          temperature=temperature),
        out_shape=[
            jax.ShapeDtypeStruct((B, NK), jnp.float32),
            jax.ShapeDtypeStruct((B, 128), jnp.float32),
        ],
    )(sims, nm2d, rmsarr)

    retrieved = pl.pallas_call(
        _stage5_body,
        grid=(nkb,),
        in_specs=[
            pl.BlockSpec((B, KB), lambda j: (0, j)),
            pl.BlockSpec((KB, D), lambda j: (j, 0)),
        ],
        out_specs=pl.BlockSpec((B, D), lambda j: (0, 0)),
        out_shape=jax.ShapeDtypeStruct((B, D), jnp.float32),
    )(weights, values)

    li = liarr[:, 0]
    x3 = x.reshape(B * S, 1, D)
    ret3 = retrieved.reshape(B, 1, D)
    gs3 = gscale.reshape(B, 1, 128)
    out3 = pl.pallas_call(
        _stage6_body,
        grid_spec=pltpu.PrefetchScalarGridSpec(
            num_scalar_prefetch=1,
            grid=(B,),
            in_specs=[
                pl.BlockSpec((1, 1, D), lambda b, li_r: (b * S + li_r[b], 0, 0)),
                pl.BlockSpec((1, 1, D), lambda b, li_r: (b, 0, 0)),
                pl.BlockSpec((1, 1, 128), lambda b, li_r: (b, 0, 0)),
            ],
            out_specs=pl.BlockSpec((1, 1, D), lambda b, li_r: (b * S + li_r[b], 0, 0)),
        ),
        out_shape=jax.ShapeDtypeStruct((B * S, 1, D), jnp.float32),
        input_output_aliases={1: 0},
    )(li, x3, ret3, gs3)

    return out3.reshape(B, S, D)


# speedup vs baseline: 1.2096x; 1.2096x over previous
import math

import jax
import jax.numpy as jnp
from jax.experimental import pallas as pl
from jax.experimental.pallas import tpu as pltpu

_TARGET_SNR = 0.3
_EPS = 1e-12


def _stage1_body(x_ref, mask_ref, qn_ref, rms_ref, li_ref):
    B, S, D = x_ref.shape
    q_rows = []
    rms_rows = []
    li_rows = []
    iota_s = jax.lax.broadcasted_iota(jnp.int32, (1, S), 1)
    for b in range(B):
        xb = x_ref[b]
        mb = mask_ref[pl.ds(b, 1), :]
        cnt = jnp.sum(mb, axis=1, keepdims=True)
        denom = jnp.maximum(cnt, 1.0)
        li_b = (jnp.maximum(cnt, 1.0) - 1.0).astype(jnp.int32)
        qsum = jax.lax.dot_general(
            mb, xb, (((1,), (0,)), ((), ())),
            preferred_element_type=jnp.float32)
        qmean = qsum / denom
        qn = qmean / jnp.maximum(
            jnp.sqrt(jnp.sum(qmean * qmean, axis=1, keepdims=True)), _EPS)
        q_rows.append(qn.astype(jnp.bfloat16))
        onehot = (iota_s == li_b).astype(jnp.float32)
        host = jax.lax.dot_general(
            onehot, xb, (((1,), (0,)), ((), ())),
            preferred_element_type=jnp.float32)
        rms = jnp.sqrt(jnp.mean(host * host, axis=1, keepdims=True))
        rms_rows.append(jnp.broadcast_to(rms, (1, 128)))
        li_rows.append(jnp.broadcast_to(li_b, (1, 128)))
    qn_ref[...] = jnp.concatenate(q_rows, axis=0)
    rms_ref[...] = jnp.concatenate(rms_rows, axis=0)
    li_ref[...] = jnp.concatenate(li_rows, axis=0)


def _stage2_body(k_ref, kn_ref):
    k = k_ref[...]
    norm = jnp.sqrt(jnp.sum(k * k, axis=1, keepdims=True))
    kn_ref[...] = (k / jnp.maximum(norm, _EPS)).astype(jnp.bfloat16)


def _stage3_body(il_ref, jl_ref, ki_ref, kj_ref, q_ref,
                 nmrow_ref, nmcol_ref, sims_ref,
                 rpart_ref, cpart_ref, *, nkb, t_total):
    t = pl.program_id(0)
    i = il_ref[t]
    j = jl_ref[t]
    g = jax.lax.dot_general(
        ki_ref[...], kj_ref[...], (((1,), (1,)), ((), ())),
        preferred_element_type=jnp.float32)
    KB = g.shape[0]
    rows = jax.lax.broadcasted_iota(jnp.int32, (KB, KB), 0)
    cols = jax.lax.broadcasted_iota(jnp.int32, (KB, KB), 1)
    on_diag = jnp.logical_and(rows == cols, i == j)
    g = jnp.where(on_diag, -jnp.inf, g)

    rp = g[:, 0:128]
    for kk in range(1, KB // 128):
        rp = jnp.maximum(rp, g[:, kk * 128:(kk + 1) * 128])
    first_of_i = i == j
    rpart_ref[...] = jnp.where(first_of_i, rp,
                               jnp.maximum(rpart_ref[...], rp))

    @pl.when(j == nkb - 1)
    def _emit_row():
        nmrow_ref[...] = jnp.max(rpart_ref[...], axis=1, keepdims=True)

    cm = jnp.max(g, axis=0, keepdims=True)

    @pl.when(t == 0)
    def _init_cpart():
        cpart_ref[...] = jnp.full((nkb, KB), -jnp.inf, jnp.float32)

    rows8 = jax.lax.broadcasted_iota(jnp.int32, (nkb, 1), 0)
    upd = jnp.where(rows8 == j, jnp.broadcast_to(cm, (nkb, KB)), -jnp.inf)
    cpart_ref[...] = jnp.maximum(cpart_ref[...], upd)

    @pl.when(t == t_total - 1)
    def _emit_col():
        nmcol_ref[...] = cpart_ref[...]

    @pl.when(i == 0)
    def _sims():
        sims_ref[...] = jax.lax.dot_general(
            q_ref[...], kj_ref[...], (((1,), (1,)), ((), ())),
            preferred_element_type=jnp.float32)


def _kth_largest_rows(a, k, iters=40):
    rmin = jnp.min(a, axis=1, keepdims=True)
    rmax = jnp.max(a, axis=1, keepdims=True)
    lo0 = rmin - 1.0
    hi0 = rmax + 1.0

    def body(_, carry):
        lo, hi = carry
        mid = 0.5 * (lo + hi)
        cnt = jnp.sum((a >= mid).astype(jnp.float32), axis=1, keepdims=True)
        ge = cnt >= float(k)
        return (jnp.where(ge, mid, lo), jnp.where(ge, hi, mid))

    lo, hi = jax.lax.fori_loop(0, iters, body, (lo0, hi0))
    return jnp.max(jnp.where(a < hi, a, rmin - 2.0), axis=1, keepdims=True)


def _stage4_body(sims_ref, nma_ref, nmb_ref, rms_ref, w_ref, gs_ref, *,
                 nk, d, eff_k, temperature):
    sims = sims_ref[...]
    nm = jnp.maximum(nma_ref[...], nmb_ref[...])
    sqrt_d = math.sqrt(float(d))

    mean_raw = jnp.mean(sims, axis=1, keepdims=True)
    var_raw = jnp.mean((sims - mean_raw) ** 2, axis=1, keepdims=True)
    std_raw = jnp.maximum(jnp.sqrt(var_raw), 1e-6)
    max_raw = jnp.max(sims, axis=1, keepdims=True)
    z_peak = (max_raw - mean_raw) / std_raw
    gate_peak = jax.nn.sigmoid(z_peak * sqrt_d)

    nm_flat = nm.reshape(1, -1)
    k1 = (nk + 1) // 2
    k2 = nk // 2 + 1
    v1 = _kth_largest_rows(nm_flat, k1)
    v2 = _kth_largest_rows(nm_flat, k2)
    tau = 0.5 * (v1[0, 0] + v2[0, 0])
    mu_nm = jnp.mean(nm_flat)
    sigma = jnp.maximum(jnp.sqrt(jnp.mean((nm_flat - mu_nm) ** 2)), 1e-6)
    gate_manifold = jax.nn.sigmoid((max_raw - tau) / sigma * sqrt_d)

    if eff_k < nk:
        thr = _kth_largest_rows(sims, eff_k)
        logits = jnp.where(sims >= thr, sims, -1e9)
    else:
        logits = sims
    ex = jnp.exp((logits - max_raw) / temperature)
    w_ref[...] = ex / jnp.sum(ex, axis=1, keepdims=True)

    gate = gate_peak * gate_manifold
    magnitude = rms_ref[...] * _TARGET_SNR
    gs_ref[...] = gate * magnitude


def _stage5_body(w_ref, v_ref, acc_ref):
    j = pl.program_id(0)

    @pl.when(j == 0)
    def _init():
        acc_ref[...] = jnp.zeros_like(acc_ref)

    part = jax.lax.dot_general(
        w_ref[...].astype(jnp.bfloat16), v_ref[...].astype(jnp.bfloat16),
        (((1,), (0,)), ((), ())), preferred_element_type=jnp.float32)
    acc_ref[...] = acc_ref[...] + part


def _stage6_body(li_ref, x_ref, ret_ref, gs_ref, out_ref):
    r = ret_ref[...]
    norm = jnp.sqrt(jnp.sum(r * r, axis=2, keepdims=True))
    dirn = r / jnp.maximum(norm, _EPS)
    out_ref[...] = x_ref[...] + gs_ref[0, 0, 0] * dirn


def kernel(x, attention_mask, keys, values):
    B, S, D = x.shape
    NK = keys.shape[0]
    temperature = math.sqrt(math.log1p(float(NK))) / max(float(D), 1.0)
    eff_k = min(NK, max(2, int(math.ceil(
        math.sqrt(float(NK)) * math.log1p(float(NK))))))

    mask_f = attention_mask.astype(jnp.float32)

    qn, rmsarr, liarr = pl.pallas_call(
        _stage1_body,
        out_shape=[
            jax.ShapeDtypeStruct((B, D), jnp.bfloat16),
            jax.ShapeDtypeStruct((B, 128), jnp.float32),
            jax.ShapeDtypeStruct((B, 128), jnp.int32),
        ],
    )(x, mask_f)

    KB = 1024
    nkb = NK // KB
    kn = pl.pallas_call(
        _stage2_body,
        grid=(nkb,),
        in_specs=[pl.BlockSpec((KB, D), lambda i: (i, 0))],
        out_specs=pl.BlockSpec((KB, D), lambda i: (i, 0)),
        out_shape=jax.ShapeDtypeStruct((NK, D), jnp.bfloat16),
    )(keys)

    import functools
    pairs = [(i, j) for i in range(nkb) for j in range(i, nkb)]
    t_total = len(pairs)
    il = jnp.asarray([p[0] for p in pairs], dtype=jnp.int32)
    jl = jnp.asarray([p[1] for p in pairs], dtype=jnp.int32)
    nmrow, nmcol, sims = pl.pallas_call(
        functools.partial(_stage3_body, nkb=nkb, t_total=t_total),
        grid_spec=pltpu.PrefetchScalarGridSpec(
            num_scalar_prefetch=2,
            grid=(t_total,),
            in_specs=[
                pl.BlockSpec((KB, D), lambda t, il_r, jl_r: (il_r[t], 0)),
                pl.BlockSpec((KB, D), lambda t, il_r, jl_r: (jl_r[t], 0)),
                pl.BlockSpec((B, D), lambda t, il_r, jl_r: (0, 0)),
            ],
            out_specs=[
                pl.BlockSpec((KB, 1), lambda t, il_r, jl_r: (il_r[t], 0)),
                pl.BlockSpec((nkb, KB), lambda t, il_r, jl_r: (0, 0)),
                pl.BlockSpec((B, KB), lambda t, il_r, jl_r: (0, jl_r[t])),
            ],
            scratch_shapes=[
                pltpu.VMEM((KB, 128), jnp.float32),
                pltpu.VMEM((nkb, KB), jnp.float32),
            ],
        ),
        out_shape=[
            jax.ShapeDtypeStruct((NK, 1), jnp.float32),
            jax.ShapeDtypeStruct((nkb, KB), jnp.float32),
            jax.ShapeDtypeStruct((B, NK), jnp.float32),
        ],
    )(il, jl, kn, kn, qn)

    nma = nmrow.reshape(NK // 128, 128)
    nmb = nmcol.reshape(NK // 128, 128)
    weights, gscale = pl.pallas_call(
        functools.partial(_stage4_body, nk=NK, d=D, eff_k=eff_k,
                          temperature=temperature),
        out_shape=[
            jax.ShapeDtypeStruct((B, NK), jnp.float32),
            jax.ShapeDtypeStruct((B, 128), jnp.float32),
        ],
    )(sims, nma, nmb, rmsarr)

    retrieved = pl.pallas_call(
        _stage5_body,
        grid=(nkb,),
        in_specs=[
            pl.BlockSpec((B, KB), lambda j: (0, j)),
            pl.BlockSpec((KB, D), lambda j: (j, 0)),
        ],
        out_specs=pl.BlockSpec((B, D), lambda j: (0, 0)),
        out_shape=jax.ShapeDtypeStruct((B, D), jnp.float32),
    )(weights, values)

    li = liarr[:, 0]
    x3 = x.reshape(B * S, 1, D)
    ret3 = retrieved.reshape(B, 1, D)
    gs3 = gscale.reshape(B, 1, 128)
    out3 = pl.pallas_call(
        _stage6_body,
        grid_spec=pltpu.PrefetchScalarGridSpec(
            num_scalar_prefetch=1,
            grid=(B,),
            in_specs=[
                pl.BlockSpec((1, 1, D), lambda b, li_r: (b * S + li_r[b], 0, 0)),
                pl.BlockSpec((1, 1, D), lambda b, li_r: (b, 0, 0)),
                pl.BlockSpec((1, 1, 128), lambda b, li_r: (b, 0, 0)),
            ],
            out_specs=pl.BlockSpec((1, 1, D), lambda b, li_r: (b * S + li_r[b], 0, 0)),
        ),
        out_shape=jax.ShapeDtypeStruct((B * S, 1, D), jnp.float32),
        input_output_aliases={1: 0},
    )(li, x3, ret3, gs3)

    return out3.reshape(B, S, D)


# speedup vs baseline: 1.2369x; 1.0225x over previous
import math

import jax
import jax.numpy as jnp
from jax.experimental import pallas as pl
from jax.experimental.pallas import tpu as pltpu

_TARGET_SNR = 0.3
_EPS = 1e-12


def _stage1_body(x_ref, mask_ref, qn_ref, rms_ref, li_ref):
    B, S, D = x_ref.shape
    q_rows = []
    rms_rows = []
    li_rows = []
    iota_s = jax.lax.broadcasted_iota(jnp.int32, (1, S), 1)
    for b in range(B):
        xb = x_ref[b]
        mb = mask_ref[pl.ds(b, 1), :]
        cnt = jnp.sum(mb, axis=1, keepdims=True)
        denom = jnp.maximum(cnt, 1.0)
        li_b = (jnp.maximum(cnt, 1.0) - 1.0).astype(jnp.int32)
        qsum = jax.lax.dot_general(
            mb, xb, (((1,), (0,)), ((), ())),
            preferred_element_type=jnp.float32)
        qmean = qsum / denom
        qn = qmean / jnp.maximum(
            jnp.sqrt(jnp.sum(qmean * qmean, axis=1, keepdims=True)), _EPS)
        q_rows.append(qn.astype(jnp.bfloat16))
        onehot = (iota_s == li_b).astype(jnp.float32)
        host = jax.lax.dot_general(
            onehot, xb, (((1,), (0,)), ((), ())),
            preferred_element_type=jnp.float32)
        rms = jnp.sqrt(jnp.mean(host * host, axis=1, keepdims=True))
        rms_rows.append(jnp.broadcast_to(rms, (1, 128)))
        li_rows.append(jnp.broadcast_to(li_b, (1, 128)))
    qn_ref[...] = jnp.concatenate(q_rows, axis=0)
    rms_ref[...] = jnp.concatenate(rms_rows, axis=0)
    li_ref[...] = jnp.concatenate(li_rows, axis=0)


def _stage2_body(k_ref, q_ref, kn_ref, sims_ref):
    k = k_ref[...]
    norm = jnp.sqrt(jnp.sum(k * k, axis=1, keepdims=True))
    knb = (k / jnp.maximum(norm, _EPS)).astype(jnp.bfloat16)
    kn_ref[...] = knb
    sims_ref[...] = jax.lax.dot_general(
        q_ref[...], knb, (((1,), (1,)), ((), ())),
        preferred_element_type=jnp.float32)


def _stage3_body(il_ref, jl_ref, ki_ref, kj_ref,
                 nmrow_ref, nmcol_ref,
                 rpart_ref, cpart_ref, *, nkb, t_total, strip):
    t = pl.program_id(0)
    i = il_ref[t]
    j = jl_ref[t]
    ki = ki_ref[...]
    KB = ki.shape[0]
    first_of_i = i == j

    @pl.when(t == 0)
    def _init_cpart():
        cpart_ref[...] = jnp.full((nkb, KB), -jnp.inf, jnp.float32)

    rows8 = jax.lax.broadcasted_iota(jnp.int32, (nkb, 1), 0)
    rows = jax.lax.broadcasted_iota(jnp.int32, (KB, strip), 0)
    cols = jax.lax.broadcasted_iota(jnp.int32, (KB, strip), 1)

    rp = None
    cms = []
    for s in range(KB // strip):
        gs = jax.lax.dot_general(
            ki, kj_ref[pl.ds(s * strip, strip), :],
            (((1,), (1,)), ((), ())),
            preferred_element_type=jnp.float32)
        on_diag = jnp.logical_and(rows == cols + s * strip, i == j)
        gs = jnp.where(on_diag, -jnp.inf, gs)
        rps = gs[:, 0:128]
        for kk in range(1, strip // 128):
            rps = jnp.maximum(rps, gs[:, kk * 128:(kk + 1) * 128])
        rp = rps if rp is None else jnp.maximum(rp, rps)
        cms.append(jnp.max(gs, axis=0, keepdims=True))

    rpart_ref[...] = jnp.where(first_of_i, rp,
                               jnp.maximum(rpart_ref[...], rp))

    @pl.when(j == nkb - 1)
    def _emit_row():
        nmrow_ref[...] = jnp.max(rpart_ref[...], axis=1, keepdims=True)

    cm = jnp.concatenate(cms, axis=1)
    upd = jnp.where(rows8 == j, jnp.broadcast_to(cm, (nkb, KB)), -jnp.inf)
    cpart_ref[...] = jnp.maximum(cpart_ref[...], upd)

    @pl.when(t == t_total - 1)
    def _emit_col():
        nmcol_ref[...] = cpart_ref[...]


def _kth_largest_rows(a, k, iters=40):
    rmin = jnp.min(a, axis=1, keepdims=True)
    rmax = jnp.max(a, axis=1, keepdims=True)
    lo0 = rmin - 1.0
    hi0 = rmax + 1.0

    def body(_, carry):
        lo, hi = carry
        mid = 0.5 * (lo + hi)
        cnt = jnp.sum((a >= mid).astype(jnp.float32), axis=1, keepdims=True)
        ge = cnt >= float(k)
        return (jnp.where(ge, mid, lo), jnp.where(ge, hi, mid))

    lo, hi = jax.lax.fori_loop(0, iters, body, (lo0, hi0))
    return jnp.max(jnp.where(a < hi, a, rmin - 2.0), axis=1, keepdims=True)


def _stage4_body(sims_ref, nma_ref, nmb_ref, rms_ref, w_ref, gs_ref, *,
                 nk, d, eff_k, temperature):
    sims = sims_ref[...]
    nm = jnp.maximum(nma_ref[...], nmb_ref[...])
    sqrt_d = math.sqrt(float(d))

    mean_raw = jnp.mean(sims, axis=1, keepdims=True)
    var_raw = jnp.mean((sims - mean_raw) ** 2, axis=1, keepdims=True)
    std_raw = jnp.maximum(jnp.sqrt(var_raw), 1e-6)
    max_raw = jnp.max(sims, axis=1, keepdims=True)
    z_peak = (max_raw - mean_raw) / std_raw
    gate_peak = jax.nn.sigmoid(z_peak * sqrt_d)

    nm_flat = nm.reshape(1, -1)
    k1 = (nk + 1) // 2
    k2 = nk // 2 + 1
    v1 = _kth_largest_rows(nm_flat, k1)
    v2 = _kth_largest_rows(nm_flat, k2)
    tau = 0.5 * (v1[0, 0] + v2[0, 0])
    mu_nm = jnp.mean(nm_flat)
    sigma = jnp.maximum(jnp.sqrt(jnp.mean((nm_flat - mu_nm) ** 2)), 1e-6)
    gate_manifold = jax.nn.sigmoid((max_raw - tau) / sigma * sqrt_d)

    if eff_k < nk:
        thr = _kth_largest_rows(sims, eff_k)
        logits = jnp.where(sims >= thr, sims, -1e9)
    else:
        logits = sims
    ex = jnp.exp((logits - max_raw) / temperature)
    w_ref[...] = ex / jnp.sum(ex, axis=1, keepdims=True)

    gate = gate_peak * gate_manifold
    magnitude = rms_ref[...] * _TARGET_SNR
    gs_ref[...] = gate * magnitude


def _stage5_body(w_ref, v_ref, acc_ref):
    j = pl.program_id(0)

    @pl.when(j == 0)
    def _init():
        acc_ref[...] = jnp.zeros_like(acc_ref)

    part = jax.lax.dot_general(
        w_ref[...].astype(jnp.bfloat16), v_ref[...].astype(jnp.bfloat16),
        (((1,), (0,)), ((), ())), preferred_element_type=jnp.float32)
    acc_ref[...] = acc_ref[...] + part


def _stage6_body(li_ref, x_ref, ret_ref, gs_ref, out_ref):
    r = ret_ref[...]
    norm = jnp.sqrt(jnp.sum(r * r, axis=2, keepdims=True))
    dirn = r / jnp.maximum(norm, _EPS)
    out_ref[...] = x_ref[...] + gs_ref[0, 0, 0] * dirn


def kernel(x, attention_mask, keys, values):
    B, S, D = x.shape
    NK = keys.shape[0]
    temperature = math.sqrt(math.log1p(float(NK))) / max(float(D), 1.0)
    eff_k = min(NK, max(2, int(math.ceil(
        math.sqrt(float(NK)) * math.log1p(float(NK))))))

    mask_f = attention_mask.astype(jnp.float32)

    qn, rmsarr, liarr = pl.pallas_call(
        _stage1_body,
        out_shape=[
            jax.ShapeDtypeStruct((B, D), jnp.bfloat16),
            jax.ShapeDtypeStruct((B, 128), jnp.float32),
            jax.ShapeDtypeStruct((B, 128), jnp.int32),
        ],
    )(x, mask_f)

    KB = 1024
    nkb = NK // KB
    kn, sims = pl.pallas_call(
        _stage2_body,
        grid=(nkb,),
        in_specs=[
            pl.BlockSpec((KB, D), lambda i: (i, 0)),
            pl.BlockSpec((B, D), lambda i: (0, 0)),
        ],
        out_specs=[
            pl.BlockSpec((KB, D), lambda i: (i, 0)),
            pl.BlockSpec((B, KB), lambda i: (0, i)),
        ],
        out_shape=[
            jax.ShapeDtypeStruct((NK, D), jnp.bfloat16),
            jax.ShapeDtypeStruct((B, NK), jnp.float32),
        ],
    )(keys, qn)

    import functools
    pairs = [(i, j) for i in range(nkb) for j in range(i, nkb)]
    t_total = len(pairs)
    il = jnp.asarray([p[0] for p in pairs], dtype=jnp.int32)
    jl = jnp.asarray([p[1] for p in pairs], dtype=jnp.int32)
    nmrow, nmcol = pl.pallas_call(
        functools.partial(_stage3_body, nkb=nkb, t_total=t_total, strip=256),
        grid_spec=pltpu.PrefetchScalarGridSpec(
            num_scalar_prefetch=2,
            grid=(t_total,),
            in_specs=[
                pl.BlockSpec((KB, D), lambda t, il_r, jl_r: (il_r[t], 0)),
                pl.BlockSpec((KB, D), lambda t, il_r, jl_r: (jl_r[t], 0)),
            ],
            out_specs=[
                pl.BlockSpec((KB, 1), lambda t, il_r, jl_r: (il_r[t], 0)),
                pl.BlockSpec((nkb, KB), lambda t, il_r, jl_r: (0, 0)),
            ],
            scratch_shapes=[
                pltpu.VMEM((KB, 128), jnp.float32),
                pltpu.VMEM((nkb, KB), jnp.float32),
            ],
        ),
        out_shape=[
            jax.ShapeDtypeStruct((NK, 1), jnp.float32),
            jax.ShapeDtypeStruct((nkb, KB), jnp.float32),
        ],
    )(il, jl, kn, kn)

    nma = nmrow.reshape(NK // 128, 128)
    nmb = nmcol.reshape(NK // 128, 128)
    weights, gscale = pl.pallas_call(
        functools.partial(_stage4_body, nk=NK, d=D, eff_k=eff_k,
                          temperature=temperature),
        out_shape=[
            jax.ShapeDtypeStruct((B, NK), jnp.float32),
            jax.ShapeDtypeStruct((B, 128), jnp.float32),
        ],
    )(sims, nma, nmb, rmsarr)

    retrieved = pl.pallas_call(
        _stage5_body,
        grid=(nkb,),
        in_specs=[
            pl.BlockSpec((B, KB), lambda j: (0, j)),
            pl.BlockSpec((KB, D), lambda j: (j, 0)),
        ],
        out_specs=pl.BlockSpec((B, D), lambda j: (0, 0)),
        out_shape=jax.ShapeDtypeStruct((B, D), jnp.float32),
    )(weights, values)

    li = liarr[:, 0]
    x3 = x.reshape(B * S, 1, D)
    ret3 = retrieved.reshape(B, 1, D)
    gs3 = gscale.reshape(B, 1, 128)
    out3 = pl.pallas_call(
        _stage6_body,
        grid_spec=pltpu.PrefetchScalarGridSpec(
            num_scalar_prefetch=1,
            grid=(B,),
            in_specs=[
                pl.BlockSpec((1, 1, D), lambda b, li_r: (b * S + li_r[b], 0, 0)),
                pl.BlockSpec((1, 1, D), lambda b, li_r: (b, 0, 0)),
                pl.BlockSpec((1, 1, 128), lambda b, li_r: (b, 0, 0)),
            ],
            out_specs=pl.BlockSpec((1, 1, D), lambda b, li_r: (b * S + li_r[b], 0, 0)),
        ),
        out_shape=jax.ShapeDtypeStruct((B * S, 1, D), jnp.float32),
        input_output_aliases={1: 0},
    )(li, x3, ret3, gs3)

    return out3.reshape(B, S, D)


# speedup vs baseline: 1.2668x; 1.0242x over previous
import functools
import math

import jax
import jax.numpy as jnp
from jax.experimental import pallas as pl
from jax.experimental.pallas import tpu as pltpu

_TARGET_SNR = 0.3
_EPS = 1e-12
_ROW_ITERS = 16
_TAU_ITERS = 20


def _stage1_body(x_ref, mask_ref, out_ref, qn_ref, rms_ref, li_ref):
    _, S, D = x_ref.shape
    out_ref[...] = x_ref[...]
    xb = x_ref[0]
    mb = mask_ref[0]
    iota_s = jax.lax.broadcasted_iota(jnp.int32, (1, S), 1)
    cnt = jnp.sum(mb, axis=1, keepdims=True)
    denom = jnp.maximum(cnt, 1.0)
    li_b = (jnp.maximum(cnt, 1.0) - 1.0).astype(jnp.int32)
    qsum = jax.lax.dot_general(
        mb, xb, (((1,), (0,)), ((), ())),
        preferred_element_type=jnp.float32)
    qmean = qsum / denom
    qn = qmean / jnp.maximum(
        jnp.sqrt(jnp.sum(qmean * qmean, axis=1, keepdims=True)), _EPS)
    qn_ref[...] = qn.astype(jnp.bfloat16)[None]
    onehot = (iota_s == li_b).astype(jnp.float32)
    host = jax.lax.dot_general(
        onehot, xb, (((1,), (0,)), ((), ())),
        preferred_element_type=jnp.float32)
    rms = jnp.sqrt(jnp.mean(host * host, axis=1, keepdims=True))
    rms_ref[...] = jnp.broadcast_to(rms, (1, 128))[None]
    li_ref[...] = jnp.broadcast_to(li_b, (1, 128))[None]


def _stage2_body(k_ref, q_ref, kn_ref, sims_ref):
    k = k_ref[...]
    norm = jnp.sqrt(jnp.sum(k * k, axis=1, keepdims=True))
    knb = (k / jnp.maximum(norm, _EPS)).astype(jnp.bfloat16)
    kn_ref[...] = knb
    sims_ref[...] = jax.lax.dot_general(
        q_ref[...], knb, (((1,), (1,)), ((), ())),
        preferred_element_type=jnp.float32)


def _stage3_body(il_ref, jl_ref, ki_ref, kj_ref,
                 nmrow_ref, nmcol_ref,
                 rpart_ref, cpart_ref, *, nkb, t_total, strip):
    t = pl.program_id(0)
    i = il_ref[t]
    j = jl_ref[t]
    ki = ki_ref[...]
    KB = ki.shape[0]
    first_of_i = i == j

    @pl.when(t == 0)
    def _init_cpart():
        cpart_ref[...] = jnp.full((nkb, KB), -jnp.inf, jnp.float32)

    rows8 = jax.lax.broadcasted_iota(jnp.int32, (nkb, 1), 0)
    rows = jax.lax.broadcasted_iota(jnp.int32, (KB, strip), 0)
    cols = jax.lax.broadcasted_iota(jnp.int32, (KB, strip), 1)

    rp = None
    cms = []
    for s in range(KB // strip):
        gs = jax.lax.dot_general(
            ki, kj_ref[pl.ds(s * strip, strip), :],
            (((1,), (1,)), ((), ())),
            preferred_element_type=jnp.float32)
        on_diag = jnp.logical_and(rows == cols + s * strip, i == j)
        gs = jnp.where(on_diag, -jnp.inf, gs)
        rps = gs[:, 0:128]
        for kk in range(1, strip // 128):
            rps = jnp.maximum(rps, gs[:, kk * 128:(kk + 1) * 128])
        rp = rps if rp is None else jnp.maximum(rp, rps)
        cms.append(jnp.max(gs, axis=0, keepdims=True))

    rpart_ref[...] = jnp.where(first_of_i, rp,
                               jnp.maximum(rpart_ref[...], rp))

    @pl.when(j == nkb - 1)
    def _emit_row():
        nmrow_ref[...] = jnp.max(rpart_ref[...], axis=1, keepdims=True)

    cm = jnp.concatenate(cms, axis=1)
    upd = jnp.where(rows8 == j, jnp.broadcast_to(cm, (nkb, KB)), -jnp.inf)
    cpart_ref[...] = jnp.maximum(cpart_ref[...], upd)

    @pl.when(t == t_total - 1)
    def _emit_col():
        nmcol_ref[...] = cpart_ref[...]


def _stage4_body(sims_ref, nma_ref, nmb_ref, rms_ref, w_ref, gs_ref, *,
                 nk, d, eff_k, temperature):
    sims = sims_ref[...]
    nm = jnp.maximum(nma_ref[...], nmb_ref[...])
    sqrt_d = math.sqrt(float(d))

    mean_raw = jnp.mean(sims, axis=1, keepdims=True)
    var_raw = jnp.mean((sims - mean_raw) ** 2, axis=1, keepdims=True)
    std_raw = jnp.maximum(jnp.sqrt(var_raw), 1e-6)
    max_raw = jnp.max(sims, axis=1, keepdims=True)
    z_peak = (max_raw - mean_raw) / std_raw
    gate_peak = jax.nn.sigmoid(z_peak * sqrt_d)

    rmin_s = jnp.min(sims, axis=1, keepdims=True)
    lo_s = rmin_s - 1.0
    hi_s = max_raw + 1.0
    nm_min = jnp.min(nm)
    nm_max = jnp.max(nm)
    k1 = float((nk + 1) // 2)
    k2 = float(nk // 2 + 1)
    lo_a = lo_b = nm_min - 1.0
    hi_a = hi_b = nm_max + 1.0
    for it in range(max(_ROW_ITERS, _TAU_ITERS)):
        if it < _ROW_ITERS:
            mid_s = 0.5 * (lo_s + hi_s)
            cnt_s = jnp.sum((sims >= mid_s).astype(jnp.float32),
                            axis=1, keepdims=True)
            ge_s = cnt_s >= float(eff_k)
            lo_s = jnp.where(ge_s, mid_s, lo_s)
            hi_s = jnp.where(ge_s, hi_s, mid_s)
        if it < _TAU_ITERS:
            mid_a = 0.5 * (lo_a + hi_a)
            cnt_a = jnp.sum((nm >= mid_a).astype(jnp.float32))
            ge_a = cnt_a >= k1
            lo_a = jnp.where(ge_a, mid_a, lo_a)
            hi_a = jnp.where(ge_a, hi_a, mid_a)
            mid_b = 0.5 * (lo_b + hi_b)
            cnt_b = jnp.sum((nm >= mid_b).astype(jnp.float32))
            ge_b = cnt_b >= k2
            lo_b = jnp.where(ge_b, mid_b, lo_b)
            hi_b = jnp.where(ge_b, hi_b, mid_b)
    thr = jnp.max(jnp.where(sims < hi_s, sims, rmin_s - 2.0),
                  axis=1, keepdims=True)
    va = jnp.max(jnp.where(nm < hi_a, nm, nm_min - 2.0))
    vb = jnp.max(jnp.where(nm < hi_b, nm, nm_min - 2.0))
    tau = 0.5 * (va + vb)

    mu_nm = jnp.mean(nm)
    sigma = jnp.maximum(jnp.sqrt(jnp.mean((nm - mu_nm) ** 2)), 1e-6)
    gate_manifold = jax.nn.sigmoid((max_raw - tau) / sigma * sqrt_d)

    if eff_k < nk:
        logits = jnp.where(sims >= thr, sims, -1e9)
    else:
        logits = sims
    ex = jnp.exp((logits - max_raw) / temperature)
    w_ref[...] = ex / jnp.sum(ex, axis=1, keepdims=True)

    gate = gate_peak * gate_manifold
    magnitude = rms_ref[...] * _TARGET_SNR
    gs_ref[...] = gate * magnitude


def _stage5_body(w_ref, v_ref, acc_ref):
    j = pl.program_id(0)

    @pl.when(j == 0)
    def _init():
        acc_ref[...] = jnp.zeros_like(acc_ref)

    part = jax.lax.dot_general(
        w_ref[...].astype(jnp.bfloat16), v_ref[...].astype(jnp.bfloat16),
        (((1,), (0,)), ((), ())), preferred_element_type=jnp.float32)
    acc_ref[...] = acc_ref[...] + part


def _stage6_body(li_ref, x_ref, ret_ref, gs_ref, out_ref):
    r = ret_ref[...]
    norm = jnp.sqrt(jnp.sum(r * r, axis=2, keepdims=True))
    dirn = r / jnp.maximum(norm, _EPS)
    out_ref[...] = x_ref[...] + gs_ref[0, 0, 0] * dirn


def kernel(x, attention_mask, keys, values):
    B, S, D = x.shape
    NK = keys.shape[0]
    temperature = math.sqrt(math.log1p(float(NK))) / max(float(D), 1.0)
    eff_k = min(NK, max(2, int(math.ceil(
        math.sqrt(float(NK)) * math.log1p(float(NK))))))

    mask3 = attention_mask.astype(jnp.float32).reshape(B, 1, S)

    out0, qn3, rms3, li3 = pl.pallas_call(
        _stage1_body,
        grid=(B,),
        in_specs=[
            pl.BlockSpec((1, S, D), lambda b: (b, 0, 0)),
            pl.BlockSpec((1, 1, S), lambda b: (b, 0, 0)),
        ],
        out_specs=[
            pl.BlockSpec((1, S, D), lambda b: (b, 0, 0)),
            pl.BlockSpec((1, 1, D), lambda b: (b, 0, 0)),
            pl.BlockSpec((1, 1, 128), lambda b: (b, 0, 0)),
            pl.BlockSpec((1, 1, 128), lambda b: (b, 0, 0)),
        ],
        out_shape=[
            jax.ShapeDtypeStruct((B, S, D), jnp.float32),
            jax.ShapeDtypeStruct((B, 1, D), jnp.bfloat16),
            jax.ShapeDtypeStruct((B, 1, 128), jnp.float32),
            jax.ShapeDtypeStruct((B, 1, 128), jnp.int32),
        ],
    )(x, mask3)
    qn = qn3.reshape(B, D)
    rmsarr = rms3.reshape(B, 128)

    KB = 1024
    nkb = NK // KB
    kn, sims = pl.pallas_call(
        _stage2_body,
        grid=(nkb,),
        in_specs=[
            pl.BlockSpec((KB, D), lambda i: (i, 0)),
            pl.BlockSpec((B, D), lambda i: (0, 0)),
        ],
        out_specs=[
            pl.BlockSpec((KB, D), lambda i: (i, 0)),
            pl.BlockSpec((B, KB), lambda i: (0, i)),
        ],
        out_shape=[
            jax.ShapeDtypeStruct((NK, D), jnp.bfloat16),
            jax.ShapeDtypeStruct((B, NK), jnp.float32),
        ],
    )(keys, qn)

    pairs = [(i, j) for i in range(nkb) for j in range(i, nkb)]
    t_total = len(pairs)
    il = jnp.asarray([p[0] for p in pairs], dtype=jnp.int32)
    jl = jnp.asarray([p[1] for p in pairs], dtype=jnp.int32)
    nmrow, nmcol = pl.pallas_call(
        functools.partial(_stage3_body, nkb=nkb, t_total=t_total, strip=256),
        grid_spec=pltpu.PrefetchScalarGridSpec(
            num_scalar_prefetch=2,
            grid=(t_total,),
            in_specs=[
                pl.BlockSpec((KB, D), lambda t, il_r, jl_r: (il_r[t], 0)),
                pl.BlockSpec((KB, D), lambda t, il_r, jl_r: (jl_r[t], 0)),
            ],
            out_specs=[
                pl.BlockSpec((KB, 1), lambda t, il_r, jl_r: (il_r[t], 0)),
                pl.BlockSpec((nkb, KB), lambda t, il_r, jl_r: (0, 0)),
            ],
            scratch_shapes=[
                pltpu.VMEM((KB, 128), jnp.float32),
                pltpu.VMEM((nkb, KB), jnp.float32),
            ],
        ),
        out_shape=[
            jax.ShapeDtypeStruct((NK, 1), jnp.float32),
            jax.ShapeDtypeStruct((nkb, KB), jnp.float32),
        ],
    )(il, jl, kn, kn)

    nma = nmrow.reshape(NK // 128, 128)
    nmb = nmcol.reshape(NK // 128, 128)
    weights, gscale = pl.pallas_call(
        functools.partial(_stage4_body, nk=NK, d=D, eff_k=eff_k,
                          temperature=temperature),
        out_shape=[
            jax.ShapeDtypeStruct((B, NK), jnp.float32),
            jax.ShapeDtypeStruct((B, 128), jnp.float32),
        ],
    )(sims, nma, nmb, rmsarr)

    retrieved = pl.pallas_call(
        _stage5_body,
        grid=(nkb,),
        in_specs=[
            pl.BlockSpec((B, KB), lambda j: (0, j)),
            pl.BlockSpec((KB, D), lambda j: (j, 0)),
        ],
        out_specs=pl.BlockSpec((B, D), lambda j: (0, 0)),
        out_shape=jax.ShapeDtypeStruct((B, D), jnp.float32),
    )(weights, values)

    li = li3.reshape(B, 128)[:, 0]
    x3 = out0.reshape(B * S, 1, D)
    ret3 = retrieved.reshape(B, 1, D)
    gs3 = gscale.reshape(B, 1, 128)
    out3 = pl.pallas_call(
        _stage6_body,
        grid_spec=pltpu.PrefetchScalarGridSpec(
            num_scalar_prefetch=1,
            grid=(B,),
            in_specs=[
                pl.BlockSpec((1, 1, D), lambda b, li_r: (b * S + li_r[b], 0, 0)),
                pl.BlockSpec((1, 1, D), lambda b, li_r: (b, 0, 0)),
                pl.BlockSpec((1, 1, 128), lambda b, li_r: (b, 0, 0)),
            ],
            out_specs=pl.BlockSpec((1, 1, D), lambda b, li_r: (b * S + li_r[b], 0, 0)),
        ),
        out_shape=jax.ShapeDtypeStruct((B * S, 1, D), jnp.float32),
        input_output_aliases={1: 0},
    )(li, x3, ret3, gs3)

    return out3.reshape(B, S, D)
